# baseline (speedup 1.0000x reference)
import functools

import jax
import jax.numpy as jnp
from jax import lax
from jax.experimental import pallas as pl
from jax.experimental.pallas import tpu as pltpu
from jax.experimental.pallas import tpu_sc as plsc

NEG_SLOPE = 0.2

NP = 10240
K = 96
NW = 32
B = 4
NE = 10048
ROWS_PER_TILE = NP // 16


def _sc_aggregate(feat, tar2, src2, e0, e1, n_chunks):
    mesh = plsc.VectorSubcoreMesh(core_axis_name="c", subcore_axis_name="s")

    @functools.partial(
        pl.kernel,
        out_type=[
            jax.ShapeDtypeStruct((2 * NP, 128), jnp.float32),
            jax.ShapeDtypeStruct((2 * NP,), jnp.float32),
        ],
        mesh=mesh,
        compiler_params=pltpu.CompilerParams(needs_layout_passes=False),
        scratch_types=[
            pltpu.VMEM((2, B, K), jnp.int32),
            pltpu.VMEM((2, B, K), jnp.int32),
            pltpu.VMEM((NE,), jnp.float32),
            pltpu.VMEM((NE,), jnp.float32),
            pltpu.VMEM((2, K, 128), jnp.float32),
            pltpu.VMEM((2, K), jnp.float32),
            pltpu.VMEM_SHARED((NP, 128), jnp.float32),
            pltpu.VMEM_SHARED((NP,), jnp.float32),
            pltpu.SemaphoreType.DMA,
            pltpu.SemaphoreType.DMA,
            pltpu.SemaphoreType.DMA,
            pltpu.SemaphoreType.DMA,
            pltpu.SemaphoreType.DMA,
            pltpu.SemaphoreType.DMA,
        ],
    )
    def body(feat_h, tar_h, src_h, e0_h, e1_h, agg_h, dsum_h,
             tar_b, src_b, e0_v, e1_v, rows_v, w_v, acc_sh, dsum_sh,
             sem_r0, sem_r1, sem_s0, sem_s1, sem_w0, sem_w1):
        c = lax.axis_index("c")
        s = lax.axis_index("s")
        wid = c * 16 + s
        sem_r = [sem_r0, sem_r1]
        sem_s = [sem_s0, sem_s1]
        sem_w = [sem_w0, sem_w1]
        n_blocks = n_chunks // B

        pltpu.sync_copy(e0_h, e0_v)
        pltpu.sync_copy(e1_h, e1_v)

        zeros16 = jnp.zeros((16,), jnp.float32)

        def zrow(j, carry):
            for cc in range(8):
                rows_v[0, j, pl.ds(cc * 16, 16)] = zeros16
            return carry

        lax.fori_loop(0, 64, zrow, 0)
        for g in range(4):
            w_v[0, pl.ds(g * 16, 16)] = zeros16
        for k in range(ROWS_PER_TILE // 64):
            pltpu.sync_copy(rows_v.at[0, pl.ds(0, 64)],
                            acc_sh.at[pl.ds(s * ROWS_PER_TILE + k * 64, 64)])
            pltpu.sync_copy(w_v.at[0, pl.ds(0, 64)],
                            dsum_sh.at[pl.ds(s * ROWS_PER_TILE + k * 64, 64)])
        plsc.subcore_barrier()

        def stage(blk, slot):
            pltpu.sync_copy(tar_h.at[wid, pl.ds(blk * B, B)], tar_b.at[slot])
            pltpu.sync_copy(src_h.at[wid, pl.ds(blk * B, B)], src_b.at[slot])

        def wait_rows(buf):
            pltpu.make_async_copy(feat_h.at[pl.ds(0, K)], rows_v.at[buf],
                                  sem_r[buf]).wait()

        def wait_scatter(buf):
            pltpu.make_async_copy(feat_h.at[pl.ds(0, K)], rows_v.at[buf],
                                  sem_s[buf]).wait()

        def wait_wscatter(buf):
            pltpu.make_async_copy(e0_h.at[pl.ds(0, K)], w_v.at[buf],
                                  sem_w[buf]).wait()

        def fire(slot, j, buf, drain):
            if drain:
                wait_scatter(buf)
            src_row = src_b.at[slot, j]
            pltpu.async_copy(feat_h.at[src_row], rows_v.at[buf], sem_r[buf])

        def process(slot, j, buf, drain_w=True):
            tar_row = tar_b.at[slot, j]
            if drain_w:
                wait_wscatter(buf)
            for g in range(K // 16):
                ti = tar_b[slot, j, pl.ds(g * 16, 16)]
                si = src_b[slot, j, pl.ds(g * 16, 16)]
                logit = (plsc.load_gather(e0_v, [ti])
                         + plsc.load_gather(e1_v, [si]))
                w_v[buf, pl.ds(g * 16, 16)] = jnp.exp(
                    jnp.maximum(logit, NEG_SLOPE * logit))
            pltpu.async_copy(w_v.at[buf], dsum_sh.at[tar_row], sem_w[buf],
                             add=True)
            wait_rows(buf)

            def scale(g, carry2):
                w16 = w_v[buf, pl.ds(g * 16, 16)]
                for l in range(16):
                    wb = jnp.broadcast_to(w16[l], (16,))
                    e = g * 16 + l
                    for cc in range(8):
                        rows_v[buf, e, pl.ds(cc * 16, 16)] = (
                            rows_v[buf, e, pl.ds(cc * 16, 16)] * wb)
                return carry2

            lax.fori_loop(0, K // 16, scale, 0)
            pltpu.async_copy(rows_v.at[buf], acc_sh.at[tar_row], sem_s[buf],
                             add=True)

        stage(0, 0)
        stage(1, 1)
        fire(0, 0, 0, drain=False)
        for i in range(B):
            if i + 1 < B:
                fire(0, i + 1, (i + 1) % 2, drain=(i + 1 >= 2))
            else:
                fire(1, 0, 0, drain=True)
            process(0, i, i % 2, drain_w=(i >= 2))

        def block(blk, carry):
            slot = blk % 2

            @pl.when(blk < n_blocks - 1)
            def _():
                stage(blk + 1, (blk + 1) % 2)

            for i in range(B):
                if i + 1 < B:
                    fire(slot, i + 1, (i + 1) % 2, drain=True)
                else:
                    @pl.when(blk < n_blocks - 1)
                    def _():
                        fire((blk + 1) % 2, 0, 0, drain=True)
                process(slot, i, i % 2)
            return carry

        lax.fori_loop(1, n_blocks, block, 0)
        wait_scatter(0)
        wait_scatter(1)
        wait_wscatter(0)
        wait_wscatter(1)
        plsc.subcore_barrier()

        base = c * NP + s * ROWS_PER_TILE
        pltpu.sync_copy(acc_sh.at[pl.ds(s * ROWS_PER_TILE, ROWS_PER_TILE)],
                        agg_h.at[pl.ds(base, ROWS_PER_TILE)])
        pltpu.sync_copy(dsum_sh.at[pl.ds(s * ROWS_PER_TILE, ROWS_PER_TILE)],
                        dsum_h.at[pl.ds(base, ROWS_PER_TILE)])

    return body(feat, tar2, src2, e0, e1)


def _tc_project(x, W, b, att):
    def body(x_ref, w_ref, b_ref, att_ref, f_ref, e0_ref, e1_ref, es_ref):
        f = lax.dot_general(x_ref[...], w_ref[...],
                            (((1,), (1,)), ((), ())),
                            preferred_element_type=jnp.float32)
        f = f + b_ref[...][None, :]
        f_ref[...] = f
        att = att_ref[...]
        e0 = jnp.sum(f * att[:, 0][None, :], axis=1)
        e1 = jnp.sum(f * att[:, 1][None, :], axis=1)
        e0_ref[...] = e0
        e1_ref[...] = e1
        logit = e0 + e1
        es_ref[...] = jnp.exp(jnp.maximum(logit, NEG_SLOPE * logit))

    n = x.shape[0]
    return pl.pallas_call(
        body,
        out_shape=[
            jax.ShapeDtypeStruct((n, 128), jnp.float32),
            jax.ShapeDtypeStruct((n,), jnp.float32),
            jax.ShapeDtypeStruct((n,), jnp.float32),
            jax.ShapeDtypeStruct((n,), jnp.float32),
        ],
    )(x, W, b, att)


def _tc_combine(f, es, agg, dsum):
    def body(f_ref, es_ref, agg_ref, ds_ref, out_ref):
        aggs = agg_ref[pl.ds(0, NP), :] + agg_ref[pl.ds(NP, NP), :]
        denom = ds_ref[pl.ds(0, NP)] + ds_ref[pl.ds(NP, NP)] + es_ref[...]
        es = es_ref[...]
        out_ref[...] = (es[:, None] * f_ref[...] + aggs) / denom[:, None]

    return pl.pallas_call(
        body,
        out_shape=jax.ShapeDtypeStruct((NP, 128), jnp.float32),
    )(f, es, agg, dsum)


def kernel(x, edge_index, W, b, att):
    n, d = x.shape
    e = edge_index.shape[1]

    ep = ((e + NW * K * B - 1) // (NW * K * B)) * (NW * K * B)
    x_p = jnp.pad(x, ((0, NP - n), (0, 0)))
    n_chunks = ep // (NW * K)
    tar = jnp.concatenate(
        [edge_index[0], jnp.full((ep - e,), NE - 1, jnp.int32)]
    ).reshape(NW, n_chunks, K)
    src = jnp.concatenate(
        [edge_index[1], jnp.full((ep - e,), NE - 1, jnp.int32)]
    ).reshape(NW, n_chunks, K)

    feat, e0, e1, es = _tc_project(x_p, W, b, att)
    agg, dsum = _sc_aggregate(feat, tar, src, e0[:NE], e1[:NE], n_chunks)
    out = _tc_combine(feat, es, agg, dsum)
    return out[:n]


# speedup vs baseline: 1.5993x; 1.5993x over previous
import functools

import jax
import jax.numpy as jnp
from jax import lax
from jax.experimental import pallas as pl
from jax.experimental.pallas import tpu as pltpu
from jax.experimental.pallas import tpu_sc as plsc

NEG_SLOPE = 0.2

NP = 10240
K = 128
NW = 32
B = 8
ROWS_PER_TILE = NP // 16


def _sc_aggregate(feat, tar2, src2, e0, e1, n_chunks):
    mesh = plsc.VectorSubcoreMesh(core_axis_name="c", subcore_axis_name="s")

    @functools.partial(
        pl.kernel,
        out_type=[
            jax.ShapeDtypeStruct((2 * NP, 128), jnp.float32),
            jax.ShapeDtypeStruct((2 * NP,), jnp.float32),
        ],
        mesh=mesh,
        compiler_params=pltpu.CompilerParams(needs_layout_passes=False),
        scratch_types=[
            pltpu.VMEM((2, B, K), jnp.int32),
            pltpu.VMEM((2, B, K), jnp.int32),
            pltpu.VMEM((NP,), jnp.float32),
            pltpu.VMEM((2, K), jnp.float32),
            pltpu.VMEM((2, K, 128), jnp.float32),
            pltpu.VMEM((2, K), jnp.float32),
            pltpu.VMEM_SHARED((NP, 128), jnp.float32),
            pltpu.VMEM_SHARED((NP,), jnp.float32),
            pltpu.SemaphoreType.DMA,
            pltpu.SemaphoreType.DMA,
            pltpu.SemaphoreType.DMA,
            pltpu.SemaphoreType.DMA,
            pltpu.SemaphoreType.DMA,
            pltpu.SemaphoreType.DMA,
            pltpu.SemaphoreType.DMA,
            pltpu.SemaphoreType.DMA,
            pltpu.SemaphoreType.DMA,
            pltpu.SemaphoreType.DMA,
            pltpu.SemaphoreType.DMA,
            pltpu.SemaphoreType.DMA,
        ],
    )
    def body(feat_h, tar_h, src_h, e0_h, e1_h, agg_h, dsum_h,
             tar_b, src_b, e0_v, e1g, rows_v, w_v, acc_sh, dsum_sh,
             sem_r0, sem_r1, sem_e0, sem_e1, sem_s0, sem_s1, sem_w0, sem_w1,
             sem_t0, sem_t1, sem_u0, sem_u1):
        c = lax.axis_index("c")
        s = lax.axis_index("s")
        wid = c * 16 + s
        sem_r = [sem_r0, sem_r1]
        sem_e = [sem_e0, sem_e1]
        sem_s = [sem_s0, sem_s1]
        sem_w = [sem_w0, sem_w1]
        sem_t = [sem_t0, sem_t1]
        sem_u = [sem_u0, sem_u1]
        n_blocks = n_chunks // B

        pltpu.sync_copy(e0_h, e0_v)

        zeros16 = jnp.zeros((16,), jnp.float32)

        def zrow(j, carry):
            for cc in range(8):
                rows_v[0, j, pl.ds(cc * 16, 16)] = zeros16
            return carry

        lax.fori_loop(0, K, zrow, 0)
        for g in range(K // 16):
            w_v[0, pl.ds(g * 16, 16)] = zeros16
        for k in range(ROWS_PER_TILE // K):
            pltpu.sync_copy(rows_v.at[0],
                            acc_sh.at[pl.ds(s * ROWS_PER_TILE + k * K, K)])
            pltpu.sync_copy(w_v.at[0],
                            dsum_sh.at[pl.ds(s * ROWS_PER_TILE + k * K, K)])
        plsc.subcore_barrier()

        def stage(blk, slot):
            pltpu.async_copy(tar_h.at[wid, pl.ds(blk * B, B)], tar_b.at[slot],
                             sem_t[slot])
            pltpu.async_copy(src_h.at[wid, pl.ds(blk * B, B)], src_b.at[slot],
                             sem_u[slot])

        def wait_stage(slot):
            pltpu.make_async_copy(tar_h.at[0], tar_b.at[slot],
                                  sem_t[slot]).wait()
            pltpu.make_async_copy(src_h.at[0], src_b.at[slot],
                                  sem_u[slot]).wait()

        def wait_rows(buf):
            pltpu.make_async_copy(feat_h.at[pl.ds(0, K)], rows_v.at[buf],
                                  sem_r[buf]).wait()

        def wait_e1(buf):
            pltpu.make_async_copy(e1_h.at[pl.ds(0, K)], e1g.at[buf],
                                  sem_e[buf]).wait()

        def wait_scatter(buf):
            pltpu.make_async_copy(feat_h.at[pl.ds(0, K)], rows_v.at[buf],
                                  sem_s[buf]).wait()

        def wait_wscatter(buf):
            pltpu.make_async_copy(e1_h.at[pl.ds(0, K)], w_v.at[buf],
                                  sem_w[buf]).wait()

        def fire(slot, j, buf, drain):
            if drain:
                wait_scatter(buf)
            src_row = src_b.at[slot, j]
            pltpu.async_copy(feat_h.at[src_row], rows_v.at[buf], sem_r[buf])
            pltpu.async_copy(e1_h.at[src_row], e1g.at[buf], sem_e[buf])

        def process(slot, j, buf, drain_w=True):
            tar_row = tar_b.at[slot, j]
            wait_e1(buf)
            if drain_w:
                wait_wscatter(buf)
            for g in range(K // 16):
                ti = tar_b[slot, j, pl.ds(g * 16, 16)]
                logit = (plsc.load_gather(e0_v, [ti])
                         + e1g[buf, pl.ds(g * 16, 16)])
                w_v[buf, pl.ds(g * 16, 16)] = jnp.exp(
                    jnp.maximum(logit, NEG_SLOPE * logit))
            pltpu.async_copy(w_v.at[buf], dsum_sh.at[tar_row], sem_w[buf],
                             add=True)
            wait_rows(buf)

            def scale(g, carry2):
                w16 = w_v[buf, pl.ds(g * 16, 16)]
                for l in range(16):
                    wb = jnp.broadcast_to(w16[l], (16,))
                    e = g * 16 + l
                    for cc in range(8):
                        rows_v[buf, e, pl.ds(cc * 16, 16)] = (
                            rows_v[buf, e, pl.ds(cc * 16, 16)] * wb)
                return carry2

            lax.fori_loop(0, K // 16, scale, 0)
            pltpu.async_copy(rows_v.at[buf], acc_sh.at[tar_row], sem_s[buf],
                             add=True)

        stage(0, 0)
        stage(1, 1)
        wait_stage(0)
        fire(0, 0, 0, drain=False)
        for i in range(B):
            if i + 1 < B:
                fire(0, i + 1, (i + 1) % 2, drain=(i + 1 >= 2))
            else:
                wait_stage(1)
                fire(1, 0, 0, drain=True)
            process(0, i, i % 2, drain_w=(i >= 2))

        def block(blk, carry):
            slot = blk % 2
            nxt = (blk + 1) % 2

            @pl.when((blk < n_blocks - 1) & (nxt == 0))
            def _():
                stage(blk + 1, 0)

            @pl.when((blk < n_blocks - 1) & (nxt == 1))
            def _():
                stage(blk + 1, 1)

            for i in range(B):
                if i + 1 < B:
                    fire(slot, i + 1, (i + 1) % 2, drain=True)
                else:
                    @pl.when((blk < n_blocks - 1) & (nxt == 0))
                    def _():
                        wait_stage(0)
                        fire(0, 0, 0, drain=True)

                    @pl.when((blk < n_blocks - 1) & (nxt == 1))
                    def _():
                        wait_stage(1)
                        fire(1, 0, 0, drain=True)
                process(slot, i, i % 2)
            return carry

        lax.fori_loop(1, n_blocks, block, 0)
        wait_scatter(0)
        wait_scatter(1)
        wait_wscatter(0)
        wait_wscatter(1)
        plsc.subcore_barrier()

        base = c * NP + s * ROWS_PER_TILE
        pltpu.sync_copy(acc_sh.at[pl.ds(s * ROWS_PER_TILE, ROWS_PER_TILE)],
                        agg_h.at[pl.ds(base, ROWS_PER_TILE)])
        pltpu.sync_copy(dsum_sh.at[pl.ds(s * ROWS_PER_TILE, ROWS_PER_TILE)],
                        dsum_h.at[pl.ds(base, ROWS_PER_TILE)])

    return body(feat, tar2, src2, e0, e1)


def _tc_project(x, W, b, att):
    def body(x_ref, w_ref, b_ref, att_ref, f_ref, e0_ref, e1_ref, es_ref):
        f = lax.dot_general(x_ref[...], w_ref[...],
                            (((1,), (1,)), ((), ())),
                            preferred_element_type=jnp.float32)
        f = f + b_ref[...][None, :]
        f_ref[...] = f
        att = att_ref[...]
        e0 = jnp.sum(f * att[:, 0][None, :], axis=1)
        e1 = jnp.sum(f * att[:, 1][None, :], axis=1)
        e0_ref[...] = e0
        e1_ref[...] = e1
        logit = e0 + e1
        es_ref[...] = jnp.exp(jnp.maximum(logit, NEG_SLOPE * logit))

    n = x.shape[0]
    return pl.pallas_call(
        body,
        out_shape=[
            jax.ShapeDtypeStruct((n, 128), jnp.float32),
            jax.ShapeDtypeStruct((n,), jnp.float32),
            jax.ShapeDtypeStruct((n,), jnp.float32),
            jax.ShapeDtypeStruct((n,), jnp.float32),
        ],
    )(x, W, b, att)


def _tc_combine(f, es, agg, dsum):
    def body(f_ref, es_ref, agg_ref, ds_ref, out_ref):
        aggs = agg_ref[pl.ds(0, NP), :] + agg_ref[pl.ds(NP, NP), :]
        denom = ds_ref[pl.ds(0, NP)] + ds_ref[pl.ds(NP, NP)] + es_ref[...]
        es = es_ref[...]
        out_ref[...] = (es[:, None] * f_ref[...] + aggs) / denom[:, None]

    return pl.pallas_call(
        body,
        out_shape=jax.ShapeDtypeStruct((NP, 128), jnp.float32),
    )(f, es, agg, dsum)


def kernel(x, edge_index, W, b, att):
    n, d = x.shape
    e = edge_index.shape[1]

    ep = ((e + NW * K * B - 1) // (NW * K * B)) * (NW * K * B)
    x_p = jnp.pad(x, ((0, NP - n), (0, 0)))
    n_chunks = ep // (NW * K)
    tar = jnp.concatenate(
        [edge_index[0], jnp.full((ep - e,), NP - 1, jnp.int32)]
    ).reshape(NW, n_chunks, K)
    src = jnp.concatenate(
        [edge_index[1], jnp.full((ep - e,), NP - 1, jnp.int32)]
    ).reshape(NW, n_chunks, K)

    feat, e0, e1, es = _tc_project(x_p, W, b, att)
    agg, dsum = _sc_aggregate(feat, tar, src, e0, e1, n_chunks)
    out = _tc_combine(feat, es, agg, dsum)
    return out[:n]
